# Initial kernel scaffold; baseline (speedup 1.0000x reference)
import functools

import jax
import jax.numpy as jnp
from jax import lax
from jax.experimental import pallas as pl
from jax.experimental.pallas import tpu as pltpu
from jax.experimental.pallas import tpu_sc as plsc

VOCAB = 256
B = 4096
L = 50

NC = 2
NS = 16
NW = NC * NS

TOTAL = B * L
PER_W = TOTAL // NW
CHUNK = 128
NCHUNK = PER_W // CHUNK


def _sc_gather(idx_flat, W):
    mesh = plsc.VectorSubcoreMesh(core_axis_name="c", subcore_axis_name="s")

    @functools.partial(
        pl.kernel,
        mesh=mesh,
        out_type=jax.ShapeDtypeStruct((TOTAL, VOCAB), jnp.float32),
        scratch_types=[
            pltpu.VMEM((PER_W,), jnp.int32),
            pltpu.VMEM((CHUNK,), jnp.int32),
            pltpu.VMEM((CHUNK, VOCAB), jnp.float32),
            pltpu.SemaphoreType.DMA,
        ],
    )
    def k(idx_hbm, w_hbm, out_hbm, idx_ws, tri_v, rows_v, sem):
        wid = lax.axis_index("s") * NC + lax.axis_index("c")
        base = wid * PER_W
        pltpu.sync_copy(idx_hbm.at[pl.ds(base, PER_W)], idx_ws)

        def chunk_body(c, carry):
            off = c * CHUNK
            lanes = lax.iota(jnp.int32, 16)
            for t in range(CHUNK // 16):
                wv = lanes + (off + t * 16)
                cur = plsc.load_gather(idx_ws, [wv])
                prv = plsc.load_gather(idx_ws, [jnp.maximum(wv - 1, 0)])
                tri = jnp.where(lax.rem(wv, L) == 0, 0, prv * VOCAB + cur)
                tri_v[pl.ds(t * 16, 16)] = tri
            pltpu.async_copy(w_hbm.at[tri_v], rows_v, sem).wait()
            pltpu.sync_copy(rows_v, out_hbm.at[pl.ds(base + off, CHUNK)])
            return carry

        lax.fori_loop(0, NCHUNK, chunk_body, 0)

    return k(idx_flat, W)


def kernel(idx, W):
    idx_flat = idx.reshape(-1).astype(jnp.int32)
    out = _sc_gather(idx_flat, W)
    return out.reshape(B, L, VOCAB)


# speedup vs baseline: 1.1921x; 1.1921x over previous
import functools

import jax
import jax.numpy as jnp
from jax import lax
from jax.experimental import pallas as pl
from jax.experimental.pallas import tpu as pltpu
from jax.experimental.pallas import tpu_sc as plsc

VOCAB = 256
B = 4096
L = 50

NC = 2
NS = 16
NW = NC * NS

TOTAL = B * L
PER_W = TOTAL // NW
CHUNK = 128
NCHUNK = PER_W // CHUNK


def _sc_gather(idx_flat, W):
    mesh = plsc.VectorSubcoreMesh(core_axis_name="c", subcore_axis_name="s")

    @functools.partial(
        pl.kernel,
        mesh=mesh,
        out_type=jax.ShapeDtypeStruct((TOTAL, VOCAB), jnp.float32),
        scratch_types=[
            pltpu.VMEM((8 + PER_W,), jnp.int32),
            pltpu.VMEM((CHUNK,), jnp.int32),
            pltpu.VMEM((CHUNK, VOCAB), jnp.float32),
            pltpu.SemaphoreType.DMA,
        ],
    )
    def k(idx_hbm, w_hbm, out_hbm, idx_ws, tri_v, rows_v, sem):
        wid = lax.axis_index("s") * NC + lax.axis_index("c")
        base = wid * PER_W
        pltpu.sync_copy(idx_hbm.at[pl.ds(base, PER_W)], idx_ws.at[pl.ds(8, PER_W)])

        def chunk_body(c, carry):
            off = c * CHUNK
            lanes = lax.iota(jnp.int32, 16)
            for t in range(CHUNK // 16):
                pos = off + t * 16
                wv = lanes + pos
                cur = idx_ws[pl.ds(8 + pos, 16)]
                prv = idx_ws[pl.ds(7 + pos, 16)]
                tri = jnp.where(lax.rem(wv, L) == 0, 0, prv * VOCAB + cur)
                tri_v[pl.ds(t * 16, 16)] = tri
            pltpu.async_copy(w_hbm.at[tri_v], rows_v, sem).wait()
            pltpu.sync_copy(rows_v, out_hbm.at[pl.ds(base + off, CHUNK)])
            return carry

        lax.fori_loop(0, NCHUNK, chunk_body, 0)

    return k(idx_flat, W)


def kernel(idx, W):
    idx_flat = idx.reshape(-1).astype(jnp.int32)
    out = _sc_gather(idx_flat, W)
    return out.reshape(B, L, VOCAB)
